# Initial kernel scaffold; baseline (speedup 1.0000x reference)
import jax
import jax.numpy as jnp
from jax.experimental import pallas as pl


def kernel(edge_features, edge_index, w2, b2, w3, b3, w4, b4, w5, b5, w6, b6, w7, b7, w8, b8, w9, b9, we, be):
    raise NotImplementedError("write your pallas kernel here")


# speedup vs baseline: 3.6363x; 3.6363x over previous
import functools

import jax
import jax.numpy as jnp
from jax import lax
from jax.experimental import pallas as pl
from jax.experimental.pallas import tpu as pltpu
from jax.experimental.pallas import tpu_sc as plsc

N_NODES = 10000
NPAD = 10240
E = 160000
C = 256
H = 128
NC = 2
NS = 16
LANES = 16

EPT = E // NS
CHK = 80
NCHK = EPT // CHK
NROWS_T = NPAD // NS

EPT32 = E // (NC * NS)
CHKF = 40
NCHKF = EPT32 // CHKF

_mesh = plsc.VectorSubcoreMesh(core_axis_name="c", subcore_axis_name="s")
_sc_params = pltpu.CompilerParams(use_tc_tiling_on_sc=False)


def _zero_rows(zbuf, shared, base, nrows, width):
    def zrow(i, _):
        def zcol(j, _):
            zbuf[i, pl.ds(j * LANES, LANES)] = jnp.zeros((LANES,), jnp.float32)
            return ()
        lax.fori_loop(0, width // LANES, zcol, ())
        return ()
    lax.fori_loop(0, zbuf.shape[0], zrow, ())
    step = zbuf.shape[0]
    def cp(k, _):
        pltpu.sync_copy(zbuf, shared.at[pl.ds(base + k * step, step)])
        return ()
    lax.fori_loop(0, nrows // step, cp, ())


@functools.partial(
    pl.kernel,
    out_type=(
        jax.ShapeDtypeStruct((NC, NPAD, H), jnp.float32),
        jax.ShapeDtypeStruct((NPAD, LANES), jnp.float32),
    ),
    mesh=_mesh,
    scratch_types=[
        pltpu.VMEM_SHARED((NPAD, H), jnp.float32),
        pltpu.VMEM_SHARED((NPAD, LANES), jnp.float32),
        pltpu.VMEM((CHK,), jnp.int32),
        pltpu.VMEM((CHK, H), jnp.float32),
        pltpu.VMEM((NROWS_T, LANES), jnp.float32),
    ],
    compiler_params=_sc_params,
)
def _sc_pass0(ef_hbm, dst_hbm, s0_hbm, cnt_hbm, acc_sh, cnt_sh, idx_v, rows_v,
              aux_v):
    c = lax.axis_index("c")
    s = lax.axis_index("s")
    nbase = s * NROWS_T

    _zero_rows(rows_v, acc_sh, nbase, NROWS_T, H)
    _zero_rows(aux_v, cnt_sh, nbase, NROWS_T, LANES)
    def orow(i, _):
        aux_v[i, :] = jnp.ones((LANES,), jnp.float32)
        return ()
    lax.fori_loop(0, CHK, orow, ())
    plsc.subcore_barrier()

    tbase = s * EPT
    def chunk(j, _):
        base = tbase + j * CHK
        pltpu.sync_copy(dst_hbm.at[pl.ds(base, CHK)], idx_v)
        pltpu.sync_copy(ef_hbm.at[pl.ds(base, CHK), pl.ds(c * H, H)], rows_v)
        pltpu.sync_copy(rows_v, acc_sh.at[idx_v], add=True)
        @pl.when(c == 0)
        def _():
            pltpu.sync_copy(aux_v.at[pl.ds(0, CHK)], cnt_sh.at[idx_v], add=True)
        return ()
    lax.fori_loop(0, NCHK, chunk, ())
    plsc.subcore_barrier()

    step = CHK
    def out(k, _):
        nb = nbase + k * step
        pltpu.sync_copy(acc_sh.at[pl.ds(nb, step)], s0_hbm.at[c, pl.ds(nb, step)])
        @pl.when(c == 0)
        def _():
            pltpu.sync_copy(cnt_sh.at[pl.ds(nb, step)], cnt_hbm.at[pl.ds(nb, step)])
        return ()
    lax.fori_loop(0, NROWS_T // step, out, ())


@functools.partial(
    pl.kernel,
    out_type=jax.ShapeDtypeStruct((NC, NPAD, H), jnp.float32),
    mesh=_mesh,
    scratch_types=[
        pltpu.VMEM_SHARED((NPAD, H), jnp.float32),
        pltpu.VMEM((CHK,), jnp.int32),
        pltpu.VMEM((CHK,), jnp.int32),
        pltpu.VMEM((CHK, H), jnp.float32),
        pltpu.SemaphoreType.DMA,
    ],
    compiler_params=_sc_params,
)
def _sc_seg(xs_hbm, src_hbm, dst_hbm, t_hbm, acc_sh, didx_v, sidx_v, rows_v,
            sem):
    c = lax.axis_index("c")
    s = lax.axis_index("s")
    nbase = s * NROWS_T

    _zero_rows(rows_v, acc_sh, nbase, NROWS_T, H)
    plsc.subcore_barrier()

    tbase = s * EPT
    off = c * NPAD
    def chunk(j, _):
        base = tbase + j * CHK
        pltpu.sync_copy(src_hbm.at[pl.ds(base, CHK)], sidx_v)
        pltpu.sync_copy(dst_hbm.at[pl.ds(base, CHK)], didx_v)
        def adj(k, _):
            sl = pl.ds(k * LANES, LANES)
            sidx_v[sl] = sidx_v[sl] + off
            return ()
        lax.fori_loop(0, CHK // LANES, adj, ())
        pltpu.async_copy(xs_hbm.at[sidx_v], rows_v, sem).wait()
        pltpu.sync_copy(rows_v, acc_sh.at[didx_v], add=True)
        return ()
    lax.fori_loop(0, NCHK, chunk, ())
    plsc.subcore_barrier()

    step = CHK
    def out(k, _):
        nb = nbase + k * step
        pltpu.sync_copy(acc_sh.at[pl.ds(nb, step)], t_hbm.at[c, pl.ds(nb, step)])
        return ()
    lax.fori_loop(0, NROWS_T // step, out, ())


@functools.partial(
    pl.kernel,
    out_type=jax.ShapeDtypeStruct((E, C), jnp.float32),
    mesh=_mesh,
    scratch_types=[
        pltpu.VMEM((CHKF,), jnp.int32),
        pltpu.VMEM((CHKF,), jnp.int32),
        pltpu.VMEM((CHKF, H), jnp.float32),
        pltpu.VMEM((CHKF, H), jnp.float32),
        pltpu.VMEM((CHKF, H), jnp.float32),
        pltpu.VMEM((CHKF, H), jnp.float32),
        pltpu.VMEM((CHKF, C), jnp.float32),
        pltpu.SemaphoreType.DMA,
        pltpu.SemaphoreType.DMA,
        pltpu.SemaphoreType.DMA,
        pltpu.SemaphoreType.DMA,
    ],
    compiler_params=_sc_params,
)
def _sc_edge(y1l_hbm, y1r_hbm, y2l_hbm, y2r_hbm, src_hbm, dst_hbm, ef_hbm,
             si_v, di_v, r1l_v, r1r_v, r2l_v, r2r_v, o_v,
             sem1, sem2, sem3, sem4):
    c = lax.axis_index("c")
    s = lax.axis_index("s")
    wid = s * NC + c
    tbase = wid * EPT32

    def chunk(j, _):
        base = tbase + j * CHKF
        pltpu.sync_copy(src_hbm.at[pl.ds(base, CHKF)], si_v)
        pltpu.sync_copy(dst_hbm.at[pl.ds(base, CHKF)], di_v)
        cp1 = pltpu.async_copy(y1l_hbm.at[si_v], r1l_v, sem1)
        cp2 = pltpu.async_copy(y1r_hbm.at[si_v], r1r_v, sem2)
        cp3 = pltpu.async_copy(y2l_hbm.at[di_v], r2l_v, sem3)
        cp4 = pltpu.async_copy(y2r_hbm.at[di_v], r2r_v, sem4)
        cp1.wait()
        cp2.wait()
        cp3.wait()
        cp4.wait()
        def row(i, _):
            for jj in range(H // LANES):
                sl = pl.ds(jj * LANES, LANES)
                v = r1l_v[i, sl] + r2l_v[i, sl]
                o_v[i, sl] = jnp.maximum(v, 0.01 * v)
                v = r1r_v[i, sl] + r2r_v[i, sl]
                o_v[i, pl.ds(H + jj * LANES, LANES)] = jnp.maximum(v, 0.01 * v)
            return ()
        lax.fori_loop(0, CHKF, row, ())
        pltpu.sync_copy(o_v, ef_hbm.at[pl.ds(base, CHKF)])
        return ()
    lax.fori_loop(0, NCHKF, chunk, ())


BLK = 1024


def _tc_fin0_body(s0_ref, cnt_ref, out_ref):
    cnt = cnt_ref[:, 0:1]
    inv = 1.0 / jnp.maximum(cnt, 1.0)
    out_ref[0] = s0_ref[0] * inv
    out_ref[1] = s0_ref[1] * inv


def _tc_fin0(s0, cnt16):
    return pl.pallas_call(
        _tc_fin0_body,
        grid=(NPAD // BLK,),
        in_specs=[
            pl.BlockSpec((NC, BLK, H), lambda i: (0, i, 0)),
            pl.BlockSpec((BLK, LANES), lambda i: (i, 0)),
        ],
        out_specs=pl.BlockSpec((NC, BLK, H), lambda i: (0, i, 0)),
        out_shape=jax.ShapeDtypeStruct((NC, NPAD, H), jnp.float32),
    )(s0, cnt16)


def _tc_layer_body(x_ref, t_ref, cnt_ref, w_ref, b_ref, out_ref):
    x = jnp.concatenate([x_ref[0], x_ref[1]], axis=1)
    t = jnp.concatenate([t_ref[0], t_ref[1]], axis=1)
    cnt = cnt_ref[:, 0:1]
    inv = 1.0 / jnp.maximum(cnt, 1.0)
    agg = t * inv
    h = jnp.concatenate([x, agg - x], axis=1)
    h = jnp.dot(h, w_ref[...], preferred_element_type=jnp.float32) + b_ref[...]
    h = jnp.where(cnt > 0.0, h, 0.0)
    h = jnp.maximum(h, 0.0)
    out_ref[0] = h[:, :H]
    out_ref[1] = h[:, H:]


def _tc_layer(x, t, cnt16, w, b):
    return pl.pallas_call(
        _tc_layer_body,
        grid=(NPAD // BLK,),
        in_specs=[
            pl.BlockSpec((NC, BLK, H), lambda i: (0, i, 0)),
            pl.BlockSpec((NC, BLK, H), lambda i: (0, i, 0)),
            pl.BlockSpec((BLK, LANES), lambda i: (i, 0)),
            pl.BlockSpec((2 * C, C), lambda i: (0, 0)),
            pl.BlockSpec((C,), lambda i: (0,)),
        ],
        out_specs=pl.BlockSpec((NC, BLK, H), lambda i: (0, i, 0)),
        out_shape=jax.ShapeDtypeStruct((NC, NPAD, H), jnp.float32),
    )(x, t, cnt16, w, b)


def _tc_final_body(x_ref, we_ref, be_ref, y1l_ref, y1r_ref, y2l_ref, y2r_ref):
    x = jnp.concatenate([x_ref[0], x_ref[1]], axis=1)
    y1 = jnp.dot(x, we_ref[:C, :], preferred_element_type=jnp.float32)
    y2 = (
        jnp.dot(x, we_ref[C:, :], preferred_element_type=jnp.float32)
        + be_ref[...]
    )
    y1l_ref[...] = y1[:, :H]
    y1r_ref[...] = y1[:, H:]
    y2l_ref[...] = y2[:, :H]
    y2r_ref[...] = y2[:, H:]


def _tc_final(x, we, be):
    return pl.pallas_call(
        _tc_final_body,
        grid=(NPAD // BLK,),
        in_specs=[
            pl.BlockSpec((NC, BLK, H), lambda i: (0, i, 0)),
            pl.BlockSpec((2 * C, C), lambda i: (0, 0)),
            pl.BlockSpec((C,), lambda i: (0,)),
        ],
        out_specs=[
            pl.BlockSpec((BLK, H), lambda i: (i, 0)),
            pl.BlockSpec((BLK, H), lambda i: (i, 0)),
            pl.BlockSpec((BLK, H), lambda i: (i, 0)),
            pl.BlockSpec((BLK, H), lambda i: (i, 0)),
        ],
        out_shape=[
            jax.ShapeDtypeStruct((NPAD, H), jnp.float32),
            jax.ShapeDtypeStruct((NPAD, H), jnp.float32),
            jax.ShapeDtypeStruct((NPAD, H), jnp.float32),
            jax.ShapeDtypeStruct((NPAD, H), jnp.float32),
        ],
    )(x, we, be)


def kernel(edge_features, edge_index, w2, b2, w3, b3, w4, b4, w5, b5, w6, b6,
           w7, b7, w8, b8, w9, b9, we, be):
    src = edge_index[0]
    dst = edge_index[1]

    s0, cnt16 = _sc_pass0(edge_features, dst)
    x = _tc_fin0(s0, cnt16)

    for (w, b) in [(w2, b2), (w3, b3), (w4, b4), (w5, b5), (w6, b6), (w7, b7),
                   (w8, b8), (w9, b9)]:
        xs = x.reshape(NC * NPAD, H)
        t = _sc_seg(xs, src, dst)
        x = _tc_layer(x, t, cnt16, w, b)

    y1l, y1r, y2l, y2r = _tc_final(x, we, be)
    ef = _sc_edge(y1l, y1r, y2l, y2r, src, dst)
    side_loss = jnp.float32(0.0)
    return (ef, side_loss)


# speedup vs baseline: 5.6724x; 1.5599x over previous
import functools

import jax
import jax.numpy as jnp
from jax import lax
from jax.experimental import pallas as pl
from jax.experimental.pallas import tpu as pltpu
from jax.experimental.pallas import tpu_sc as plsc

N_NODES = 10000
NPAD = 10240
E = 160000
C = 256
H = 128
NC = 2
NS = 16
LANES = 16

EPT = E // NS
CHK = 80
NCHK = EPT // CHK
NROWS_T = NPAD // NS

EPT32 = E // (NC * NS)
CHKF = 40
NCHKF = EPT32 // CHKF

_mesh = plsc.VectorSubcoreMesh(core_axis_name="c", subcore_axis_name="s")
_sc_params = pltpu.CompilerParams(use_tc_tiling_on_sc=False)


def _zero_rows(zbuf, shared, base, nrows, width):
    def zrow(i, _):
        def zcol(j, _):
            zbuf[i, pl.ds(j * LANES, LANES)] = jnp.zeros((LANES,), jnp.float32)
            return ()
        lax.fori_loop(0, width // LANES, zcol, ())
        return ()
    lax.fori_loop(0, zbuf.shape[0], zrow, ())
    step = zbuf.shape[0]
    def cp(k, _):
        pltpu.sync_copy(zbuf, shared.at[pl.ds(base + k * step, step)])
        return ()
    lax.fori_loop(0, nrows // step, cp, ())


@functools.partial(
    pl.kernel,
    out_type=(
        jax.ShapeDtypeStruct((NC, NPAD, H), jnp.float32),
        jax.ShapeDtypeStruct((NPAD, LANES), jnp.float32),
    ),
    mesh=_mesh,
    scratch_types=[
        pltpu.VMEM_SHARED((NPAD, H), jnp.float32),
        pltpu.VMEM_SHARED((NPAD, LANES), jnp.float32),
        pltpu.VMEM((CHK,), jnp.int32),
        pltpu.VMEM((CHK, H), jnp.float32),
        pltpu.VMEM((NROWS_T, LANES), jnp.float32),
    ],
    compiler_params=_sc_params,
)
def _sc_pass0(ef_hbm, dst_hbm, s0_hbm, cnt_hbm, acc_sh, cnt_sh, idx_v, rows_v,
              aux_v):
    c = lax.axis_index("c")
    s = lax.axis_index("s")
    nbase = s * NROWS_T

    _zero_rows(rows_v, acc_sh, nbase, NROWS_T, H)
    _zero_rows(aux_v, cnt_sh, nbase, NROWS_T, LANES)
    def orow(i, _):
        aux_v[i, :] = jnp.ones((LANES,), jnp.float32)
        return ()
    lax.fori_loop(0, CHK, orow, ())
    plsc.subcore_barrier()

    tbase = s * EPT
    def chunk(j, _):
        base = tbase + j * CHK
        pltpu.sync_copy(dst_hbm.at[pl.ds(base, CHK)], idx_v)
        pltpu.sync_copy(ef_hbm.at[pl.ds(base, CHK), pl.ds(c * H, H)], rows_v)
        pltpu.sync_copy(rows_v, acc_sh.at[idx_v], add=True)
        @pl.when(c == 0)
        def _():
            pltpu.sync_copy(aux_v.at[pl.ds(0, CHK)], cnt_sh.at[idx_v], add=True)
        return ()
    lax.fori_loop(0, NCHK, chunk, ())
    plsc.subcore_barrier()

    step = CHK
    def out(k, _):
        nb = nbase + k * step
        pltpu.sync_copy(acc_sh.at[pl.ds(nb, step)], s0_hbm.at[c, pl.ds(nb, step)])
        @pl.when(c == 0)
        def _():
            pltpu.sync_copy(cnt_sh.at[pl.ds(nb, step)], cnt_hbm.at[pl.ds(nb, step)])
        return ()
    lax.fori_loop(0, NROWS_T // step, out, ())


@functools.partial(
    pl.kernel,
    out_type=jax.ShapeDtypeStruct((NC, NPAD, H), jnp.float32),
    mesh=_mesh,
    scratch_types=[
        pltpu.VMEM_SHARED((NPAD, H), jnp.float32),
        pltpu.VMEM((EPT,), jnp.int32),
        pltpu.VMEM((EPT,), jnp.int32),
        pltpu.VMEM((CHK,), jnp.int32),
        pltpu.VMEM((2, CHK, H), jnp.float32),
        pltpu.SemaphoreType.DMA((2,)),
    ],
    compiler_params=_sc_params,
)
def _sc_seg(xs_hbm, src_hbm, dst_hbm, t_hbm, acc_sh, sidx_v, didx_v, dwork_v,
            rows_v, gsem):
    c = lax.axis_index("c")
    s = lax.axis_index("s")
    nbase = s * NROWS_T

    _zero_rows(rows_v.at[0], acc_sh, nbase, NROWS_T, H)

    tbase = s * EPT
    off = c * NPAD
    pltpu.sync_copy(src_hbm.at[pl.ds(tbase, EPT)], sidx_v)
    pltpu.sync_copy(dst_hbm.at[pl.ds(tbase, EPT)], didx_v)
    def adj(k, _):
        sl = pl.ds(k * LANES, LANES)
        sidx_v[sl] = sidx_v[sl] + off
        return ()
    lax.fori_loop(0, EPT // LANES, adj, ())
    plsc.subcore_barrier()

    def gather(j, b):
        return pltpu.make_async_copy(
            xs_hbm.at[sidx_v.at[pl.ds(j * CHK, CHK)]], rows_v.at[b],
            gsem.at[b])

    gather(0, 0).start()
    def chunk(j, _):
        b = j % 2
        @pl.when(j + 1 < NCHK)
        def _():
            gather(j + 1, (j + 1) % 2).start()
        gather(j, b).wait()
        def ld(k, _):
            sl = pl.ds(k * LANES, LANES)
            dwork_v[sl] = didx_v[pl.ds(j * CHK + k * LANES, LANES)]
            return ()
        lax.fori_loop(0, CHK // LANES, ld, ())
        pltpu.sync_copy(rows_v.at[b], acc_sh.at[dwork_v], add=True)
        return ()
    lax.fori_loop(0, NCHK, chunk, ())
    plsc.subcore_barrier()

    step = CHK
    def out(k, _):
        nb = nbase + k * step
        pltpu.sync_copy(acc_sh.at[pl.ds(nb, step)], t_hbm.at[c, pl.ds(nb, step)])
        return ()
    lax.fori_loop(0, NROWS_T // step, out, ())


@functools.partial(
    pl.kernel,
    out_type=jax.ShapeDtypeStruct((E, C), jnp.float32),
    mesh=_mesh,
    scratch_types=[
        pltpu.VMEM((CHKF,), jnp.int32),
        pltpu.VMEM((CHKF,), jnp.int32),
        pltpu.VMEM((CHKF, H), jnp.float32),
        pltpu.VMEM((CHKF, H), jnp.float32),
        pltpu.VMEM((CHKF, H), jnp.float32),
        pltpu.VMEM((CHKF, H), jnp.float32),
        pltpu.VMEM((CHKF, C), jnp.float32),
        pltpu.SemaphoreType.DMA,
        pltpu.SemaphoreType.DMA,
        pltpu.SemaphoreType.DMA,
        pltpu.SemaphoreType.DMA,
    ],
    compiler_params=_sc_params,
)
def _sc_edge(y1l_hbm, y1r_hbm, y2l_hbm, y2r_hbm, src_hbm, dst_hbm, ef_hbm,
             si_v, di_v, r1l_v, r1r_v, r2l_v, r2r_v, o_v,
             sem1, sem2, sem3, sem4):
    c = lax.axis_index("c")
    s = lax.axis_index("s")
    wid = s * NC + c
    tbase = wid * EPT32

    def chunk(j, _):
        base = tbase + j * CHKF
        pltpu.sync_copy(src_hbm.at[pl.ds(base, CHKF)], si_v)
        pltpu.sync_copy(dst_hbm.at[pl.ds(base, CHKF)], di_v)
        cp1 = pltpu.async_copy(y1l_hbm.at[si_v], r1l_v, sem1)
        cp2 = pltpu.async_copy(y1r_hbm.at[si_v], r1r_v, sem2)
        cp3 = pltpu.async_copy(y2l_hbm.at[di_v], r2l_v, sem3)
        cp4 = pltpu.async_copy(y2r_hbm.at[di_v], r2r_v, sem4)
        cp1.wait()
        cp2.wait()
        cp3.wait()
        cp4.wait()
        def row(i, _):
            for jj in range(H // LANES):
                sl = pl.ds(jj * LANES, LANES)
                v = r1l_v[i, sl] + r2l_v[i, sl]
                o_v[i, sl] = jnp.maximum(v, 0.01 * v)
                v = r1r_v[i, sl] + r2r_v[i, sl]
                o_v[i, pl.ds(H + jj * LANES, LANES)] = jnp.maximum(v, 0.01 * v)
            return ()
        lax.fori_loop(0, CHKF, row, ())
        pltpu.sync_copy(o_v, ef_hbm.at[pl.ds(base, CHKF)])
        return ()
    lax.fori_loop(0, NCHKF, chunk, ())


BLK = 1024


def _tc_fin0_body(s0_ref, cnt_ref, out_ref):
    cnt = cnt_ref[:, 0:1]
    inv = 1.0 / jnp.maximum(cnt, 1.0)
    out_ref[0] = s0_ref[0] * inv
    out_ref[1] = s0_ref[1] * inv


def _tc_fin0(s0, cnt16):
    return pl.pallas_call(
        _tc_fin0_body,
        grid=(NPAD // BLK,),
        in_specs=[
            pl.BlockSpec((NC, BLK, H), lambda i: (0, i, 0)),
            pl.BlockSpec((BLK, LANES), lambda i: (i, 0)),
        ],
        out_specs=pl.BlockSpec((NC, BLK, H), lambda i: (0, i, 0)),
        out_shape=jax.ShapeDtypeStruct((NC, NPAD, H), jnp.float32),
    )(s0, cnt16)


def _tc_layer_body(x_ref, t_ref, cnt_ref, w_ref, b_ref, out_ref):
    x = jnp.concatenate([x_ref[0], x_ref[1]], axis=1)
    t = jnp.concatenate([t_ref[0], t_ref[1]], axis=1)
    cnt = cnt_ref[:, 0:1]
    inv = 1.0 / jnp.maximum(cnt, 1.0)
    v = t * inv - x
    wt = w_ref[:C, :].astype(jnp.bfloat16)
    wb = w_ref[C:, :].astype(jnp.bfloat16)
    h = jnp.dot(x.astype(jnp.bfloat16), wt, preferred_element_type=jnp.float32)
    h = h + jnp.dot(v, wb.astype(jnp.float32),
                    preferred_element_type=jnp.float32,
                    precision=lax.Precision.HIGHEST)
    h = h + b_ref[...]
    h = jnp.where(cnt > 0.0, h, 0.0)
    h = jnp.maximum(h, 0.0)
    out_ref[0] = h[:, :H]
    out_ref[1] = h[:, H:]


def _tc_layer(x, t, cnt16, w, b):
    return pl.pallas_call(
        _tc_layer_body,
        grid=(NPAD // BLK,),
        in_specs=[
            pl.BlockSpec((NC, BLK, H), lambda i: (0, i, 0)),
            pl.BlockSpec((NC, BLK, H), lambda i: (0, i, 0)),
            pl.BlockSpec((BLK, LANES), lambda i: (i, 0)),
            pl.BlockSpec((2 * C, C), lambda i: (0, 0)),
            pl.BlockSpec((C,), lambda i: (0,)),
        ],
        out_specs=pl.BlockSpec((NC, BLK, H), lambda i: (0, i, 0)),
        out_shape=jax.ShapeDtypeStruct((NC, NPAD, H), jnp.float32),
    )(x, t, cnt16, w, b)


def _tc_final_body(x_ref, we_ref, be_ref, y1l_ref, y1r_ref, y2l_ref, y2r_ref):
    x = jnp.concatenate([x_ref[0], x_ref[1]], axis=1).astype(jnp.bfloat16)
    y1 = jnp.dot(x, we_ref[:C, :].astype(jnp.bfloat16),
                 preferred_element_type=jnp.float32)
    y2 = (
        jnp.dot(x, we_ref[C:, :].astype(jnp.bfloat16),
                preferred_element_type=jnp.float32)
        + be_ref[...]
    )
    y1l_ref[...] = y1[:, :H]
    y1r_ref[...] = y1[:, H:]
    y2l_ref[...] = y2[:, :H]
    y2r_ref[...] = y2[:, H:]


def _tc_final(x, we, be):
    return pl.pallas_call(
        _tc_final_body,
        grid=(NPAD // BLK,),
        in_specs=[
            pl.BlockSpec((NC, BLK, H), lambda i: (0, i, 0)),
            pl.BlockSpec((2 * C, C), lambda i: (0, 0)),
            pl.BlockSpec((C,), lambda i: (0,)),
        ],
        out_specs=[
            pl.BlockSpec((BLK, H), lambda i: (i, 0)),
            pl.BlockSpec((BLK, H), lambda i: (i, 0)),
            pl.BlockSpec((BLK, H), lambda i: (i, 0)),
            pl.BlockSpec((BLK, H), lambda i: (i, 0)),
        ],
        out_shape=[
            jax.ShapeDtypeStruct((NPAD, H), jnp.float32),
            jax.ShapeDtypeStruct((NPAD, H), jnp.float32),
            jax.ShapeDtypeStruct((NPAD, H), jnp.float32),
            jax.ShapeDtypeStruct((NPAD, H), jnp.float32),
        ],
    )(x, we, be)


def kernel(edge_features, edge_index, w2, b2, w3, b3, w4, b4, w5, b5, w6, b6,
           w7, b7, w8, b8, w9, b9, we, be):
    src = edge_index[0]
    dst = edge_index[1]

    s0, cnt16 = _sc_pass0(edge_features, dst)
    x = _tc_fin0(s0, cnt16)

    for (w, b) in [(w2, b2), (w3, b3), (w4, b4), (w5, b5), (w6, b6), (w7, b7),
                   (w8, b8), (w9, b9)]:
        xs = x.reshape(NC * NPAD, H)
        t = _sc_seg(xs, src, dst)
        x = _tc_layer(x, t, cnt16, w, b)

    y1l, y1r, y2l, y2r = _tc_final(x, we, be)
    ef = _sc_edge(y1l, y1r, y2l, y2r, src, dst)
    side_loss = jnp.float32(0.0)
    return (ef, side_loss)


# speedup vs baseline: 6.8052x; 1.1997x over previous
import functools

import jax
import jax.numpy as jnp
from jax import lax
from jax.experimental import pallas as pl
from jax.experimental.pallas import tpu as pltpu
from jax.experimental.pallas import tpu_sc as plsc

N_NODES = 10000
NPAD = 10240
E = 160000
C = 256
H = 128
NC = 2
NS = 16
LANES = 16

EPT = E // NS
CHK = 80
NCHK = EPT // CHK
NROWS_T = NPAD // NS

NROWS_ACC = N_NODES // NS
ZSTEP = 25
WSTEP = 125

EPT32 = E // (NC * NS)
CHKF = 40
NCHKF = EPT32 // CHKF

_mesh = plsc.VectorSubcoreMesh(core_axis_name="c", subcore_axis_name="s")
_sc_params = pltpu.CompilerParams(use_tc_tiling_on_sc=False)


def _zero_rows(zbuf, shared, base, nrows, width):
    def zrow(i, _):
        def zcol(j, _):
            zbuf[i, pl.ds(j * LANES, LANES)] = jnp.zeros((LANES,), jnp.float32)
            return ()
        lax.fori_loop(0, width // LANES, zcol, ())
        return ()
    lax.fori_loop(0, zbuf.shape[0], zrow, ())
    step = zbuf.shape[0]
    def cp(k, _):
        pltpu.sync_copy(zbuf, shared.at[pl.ds(base + k * step, step)])
        return ()
    lax.fori_loop(0, nrows // step, cp, ())


@functools.partial(
    pl.kernel,
    out_type=(
        jax.ShapeDtypeStruct((NC, NPAD, H), jnp.float32),
        jax.ShapeDtypeStruct((NPAD, LANES), jnp.float32),
    ),
    mesh=_mesh,
    scratch_types=[
        pltpu.VMEM_SHARED((N_NODES, H), jnp.float32),
        pltpu.VMEM_SHARED((N_NODES, LANES), jnp.float32),
        pltpu.VMEM((EPT,), jnp.int32),
        pltpu.VMEM((CHK,), jnp.int32),
        pltpu.VMEM((2, CHK, H), jnp.float32),
        pltpu.VMEM((ZSTEP, H), jnp.float32),
        pltpu.VMEM((WSTEP, LANES), jnp.float32),
        pltpu.SemaphoreType.DMA((2,)),
    ],
    compiler_params=_sc_params,
)
def _sc_pass0(ef_hbm, dst_hbm, s0_hbm, cnt_hbm, acc_sh, cnt_sh, didx_v,
              dwork_v, rows_v, zst_v, aux_v, lsem):
    c = lax.axis_index("c")
    s = lax.axis_index("s")
    nbase = s * NROWS_ACC

    _zero_rows(zst_v, acc_sh, nbase, NROWS_ACC, H)
    _zero_rows(aux_v, cnt_sh, nbase, NROWS_ACC, LANES)
    def orow(i, _):
        aux_v[i, :] = jnp.ones((LANES,), jnp.float32)
        return ()
    lax.fori_loop(0, CHK, orow, ())

    tbase = s * EPT
    pltpu.sync_copy(dst_hbm.at[pl.ds(tbase, EPT)], didx_v)
    plsc.subcore_barrier()

    def load(j, b):
        return pltpu.make_async_copy(
            ef_hbm.at[pl.ds(tbase + j * CHK, CHK), pl.ds(c * H, H)],
            rows_v.at[b], lsem.at[b])

    load(0, 0).start()
    def chunk(j, _):
        b = j % 2
        @pl.when(j + 1 < NCHK)
        def _():
            load(j + 1, (j + 1) % 2).start()
        load(j, b).wait()
        def ld(k, _):
            sl = pl.ds(k * LANES, LANES)
            dwork_v[sl] = didx_v[pl.ds(j * CHK + k * LANES, LANES)]
            return ()
        lax.fori_loop(0, CHK // LANES, ld, ())
        pltpu.sync_copy(rows_v.at[b], acc_sh.at[dwork_v], add=True)
        @pl.when(c == 0)
        def _():
            pltpu.sync_copy(aux_v.at[pl.ds(0, CHK)], cnt_sh.at[dwork_v],
                            add=True)
        return ()
    lax.fori_loop(0, NCHK, chunk, ())
    plsc.subcore_barrier()

    def out(k, _):
        nb = nbase + k * WSTEP
        pltpu.sync_copy(acc_sh.at[pl.ds(nb, WSTEP)],
                        s0_hbm.at[c, pl.ds(nb, WSTEP)])
        @pl.when(c == 0)
        def _():
            pltpu.sync_copy(cnt_sh.at[pl.ds(nb, WSTEP)],
                            cnt_hbm.at[pl.ds(nb, WSTEP)])
        return ()
    lax.fori_loop(0, NROWS_ACC // WSTEP, out, ())


@functools.partial(
    pl.kernel,
    out_type=jax.ShapeDtypeStruct((NC, NPAD, H), jnp.float32),
    mesh=_mesh,
    scratch_types=[
        pltpu.VMEM_SHARED((N_NODES, H), jnp.float32),
        pltpu.VMEM((ZSTEP, H), jnp.float32),
        pltpu.VMEM((EPT,), jnp.int32),
        pltpu.VMEM((EPT,), jnp.int32),
        pltpu.VMEM((CHK,), jnp.int32),
        pltpu.VMEM((2, CHK, H), jnp.float32),
        pltpu.SemaphoreType.DMA((2,)),
    ],
    compiler_params=_sc_params,
)
def _sc_seg(xs_hbm, src_hbm, dst_hbm, t_hbm, acc_sh, zst_v, sidx_v, didx_v,
            dwork_v, rows_v, gsem):
    c = lax.axis_index("c")
    s = lax.axis_index("s")
    nbase = s * NROWS_ACC

    _zero_rows(zst_v, acc_sh, nbase, NROWS_ACC, H)

    tbase = s * EPT
    off = c * NPAD
    pltpu.sync_copy(src_hbm.at[pl.ds(tbase, EPT)], sidx_v)
    pltpu.sync_copy(dst_hbm.at[pl.ds(tbase, EPT)], didx_v)
    def adj(k, _):
        sl = pl.ds(k * LANES, LANES)
        sidx_v[sl] = sidx_v[sl] + off
        return ()
    lax.fori_loop(0, EPT // LANES, adj, ())
    plsc.subcore_barrier()

    def gather(j, b):
        return pltpu.make_async_copy(
            xs_hbm.at[sidx_v.at[pl.ds(j * CHK, CHK)]], rows_v.at[b],
            gsem.at[b])

    gather(0, 0).start()
    def chunk(j, _):
        b = j % 2
        @pl.when(j + 1 < NCHK)
        def _():
            gather(j + 1, (j + 1) % 2).start()
        gather(j, b).wait()
        def ld(k, _):
            sl = pl.ds(k * LANES, LANES)
            dwork_v[sl] = didx_v[pl.ds(j * CHK + k * LANES, LANES)]
            return ()
        lax.fori_loop(0, CHK // LANES, ld, ())
        pltpu.sync_copy(rows_v.at[b], acc_sh.at[dwork_v], add=True)
        return ()
    lax.fori_loop(0, NCHK, chunk, ())
    plsc.subcore_barrier()

    def out(k, _):
        nb = nbase + k * WSTEP
        pltpu.sync_copy(acc_sh.at[pl.ds(nb, WSTEP)],
                        t_hbm.at[c, pl.ds(nb, WSTEP)])
        return ()
    lax.fori_loop(0, NROWS_ACC // WSTEP, out, ())


@functools.partial(
    pl.kernel,
    out_type=jax.ShapeDtypeStruct((E, C), jnp.float32),
    mesh=_mesh,
    scratch_types=[
        pltpu.VMEM((EPT32,), jnp.int32),
        pltpu.VMEM((EPT32,), jnp.int32),
        pltpu.VMEM((2, CHKF, H), jnp.float32),
        pltpu.VMEM((2, CHKF, H), jnp.float32),
        pltpu.VMEM((2, CHKF, H), jnp.float32),
        pltpu.VMEM((2, CHKF, H), jnp.float32),
        pltpu.VMEM((2, CHKF, C), jnp.float32),
        pltpu.SemaphoreType.DMA((2,)),
        pltpu.SemaphoreType.DMA((2,)),
    ],
    compiler_params=_sc_params,
)
def _sc_edge(y1l_hbm, y1r_hbm, y2l_hbm, y2r_hbm, src_hbm, dst_hbm, ef_hbm,
             si_v, di_v, r1l_v, r1r_v, r2l_v, r2r_v, o_v, gsem, wsem):
    c = lax.axis_index("c")
    s = lax.axis_index("s")
    wid = s * NC + c
    tbase = wid * EPT32

    pltpu.sync_copy(src_hbm.at[pl.ds(tbase, EPT32)], si_v)
    pltpu.sync_copy(dst_hbm.at[pl.ds(tbase, EPT32)], di_v)

    def gathers(j, b):
        sl = pl.ds(j * CHKF, CHKF)
        return [
            pltpu.make_async_copy(y1l_hbm.at[si_v.at[sl]], r1l_v.at[b],
                                  gsem.at[b]),
            pltpu.make_async_copy(y1r_hbm.at[si_v.at[sl]], r1r_v.at[b],
                                  gsem.at[b]),
            pltpu.make_async_copy(y2l_hbm.at[di_v.at[sl]], r2l_v.at[b],
                                  gsem.at[b]),
            pltpu.make_async_copy(y2r_hbm.at[di_v.at[sl]], r2r_v.at[b],
                                  gsem.at[b]),
        ]

    def write(j, b):
        return pltpu.make_async_copy(
            o_v.at[b], ef_hbm.at[pl.ds(tbase + j * CHKF, CHKF)], wsem.at[b])

    for cp in gathers(0, 0):
        cp.start()

    def chunk(j, _):
        b = j % 2
        @pl.when(j + 1 < NCHKF)
        def _():
            for cp in gathers(j + 1, (j + 1) % 2):
                cp.start()
        for cp in gathers(j, b):
            cp.wait()
        @pl.when(j >= 2)
        def _():
            write(j - 2, b).wait()
        def row(i, _):
            for jj in range(H // LANES):
                sl = pl.ds(jj * LANES, LANES)
                v = r1l_v[b, i, sl] + r2l_v[b, i, sl]
                o_v[b, i, sl] = jnp.maximum(v, 0.01 * v)
                v = r1r_v[b, i, sl] + r2r_v[b, i, sl]
                o_v[b, i, pl.ds(H + jj * LANES, LANES)] = (
                    jnp.maximum(v, 0.01 * v))
            return ()
        lax.fori_loop(0, CHKF, row, ())
        write(j, b).start()
        return ()
    lax.fori_loop(0, NCHKF, chunk, ())
    write(NCHKF - 2, (NCHKF - 2) % 2).wait()
    write(NCHKF - 1, (NCHKF - 1) % 2).wait()


BLK = 1024


def _tc_fin0_body(s0_ref, cnt_ref, out_ref):
    cnt = cnt_ref[:, 0:1]
    inv = 1.0 / jnp.maximum(cnt, 1.0)
    out_ref[0] = s0_ref[0] * inv
    out_ref[1] = s0_ref[1] * inv


def _tc_fin0(s0, cnt16):
    return pl.pallas_call(
        _tc_fin0_body,
        grid=(NPAD // BLK,),
        in_specs=[
            pl.BlockSpec((NC, BLK, H), lambda i: (0, i, 0)),
            pl.BlockSpec((BLK, LANES), lambda i: (i, 0)),
        ],
        out_specs=pl.BlockSpec((NC, BLK, H), lambda i: (0, i, 0)),
        out_shape=jax.ShapeDtypeStruct((NC, NPAD, H), jnp.float32),
    )(s0, cnt16)


def _tc_layer_body(x_ref, t_ref, cnt_ref, w_ref, b_ref, out_ref):
    x = jnp.concatenate([x_ref[0], x_ref[1]], axis=1)
    t = jnp.concatenate([t_ref[0], t_ref[1]], axis=1)
    cnt = cnt_ref[:, 0:1]
    inv = 1.0 / jnp.maximum(cnt, 1.0)
    v = t * inv - x
    wt = w_ref[:C, :].astype(jnp.bfloat16)
    wb = w_ref[C:, :].astype(jnp.bfloat16)
    h = jnp.dot(x.astype(jnp.bfloat16), wt, preferred_element_type=jnp.float32)
    h = h + jnp.dot(v, wb.astype(jnp.float32),
                    preferred_element_type=jnp.float32,
                    precision=lax.Precision.HIGHEST)
    h = h + b_ref[...]
    h = jnp.where(cnt > 0.0, h, 0.0)
    h = jnp.maximum(h, 0.0)
    out_ref[0] = h[:, :H]
    out_ref[1] = h[:, H:]


def _tc_layer(x, t, cnt16, w, b):
    return pl.pallas_call(
        _tc_layer_body,
        grid=(NPAD // BLK,),
        in_specs=[
            pl.BlockSpec((NC, BLK, H), lambda i: (0, i, 0)),
            pl.BlockSpec((NC, BLK, H), lambda i: (0, i, 0)),
            pl.BlockSpec((BLK, LANES), lambda i: (i, 0)),
            pl.BlockSpec((2 * C, C), lambda i: (0, 0)),
            pl.BlockSpec((C,), lambda i: (0,)),
        ],
        out_specs=pl.BlockSpec((NC, BLK, H), lambda i: (0, i, 0)),
        out_shape=jax.ShapeDtypeStruct((NC, NPAD, H), jnp.float32),
    )(x, t, cnt16, w, b)


def _tc_final_body(x_ref, we_ref, be_ref, y1l_ref, y1r_ref, y2l_ref, y2r_ref):
    x = jnp.concatenate([x_ref[0], x_ref[1]], axis=1).astype(jnp.bfloat16)
    y1 = jnp.dot(x, we_ref[:C, :].astype(jnp.bfloat16),
                 preferred_element_type=jnp.float32)
    y2 = (
        jnp.dot(x, we_ref[C:, :].astype(jnp.bfloat16),
                preferred_element_type=jnp.float32)
        + be_ref[...]
    )
    y1l_ref[...] = y1[:, :H]
    y1r_ref[...] = y1[:, H:]
    y2l_ref[...] = y2[:, :H]
    y2r_ref[...] = y2[:, H:]


def _tc_final(x, we, be):
    return pl.pallas_call(
        _tc_final_body,
        grid=(NPAD // BLK,),
        in_specs=[
            pl.BlockSpec((NC, BLK, H), lambda i: (0, i, 0)),
            pl.BlockSpec((2 * C, C), lambda i: (0, 0)),
            pl.BlockSpec((C,), lambda i: (0,)),
        ],
        out_specs=[
            pl.BlockSpec((BLK, H), lambda i: (i, 0)),
            pl.BlockSpec((BLK, H), lambda i: (i, 0)),
            pl.BlockSpec((BLK, H), lambda i: (i, 0)),
            pl.BlockSpec((BLK, H), lambda i: (i, 0)),
        ],
        out_shape=[
            jax.ShapeDtypeStruct((NPAD, H), jnp.float32),
            jax.ShapeDtypeStruct((NPAD, H), jnp.float32),
            jax.ShapeDtypeStruct((NPAD, H), jnp.float32),
            jax.ShapeDtypeStruct((NPAD, H), jnp.float32),
        ],
    )(x, we, be)


def kernel(edge_features, edge_index, w2, b2, w3, b3, w4, b4, w5, b5, w6, b6,
           w7, b7, w8, b8, w9, b9, we, be):
    src = edge_index[0]
    dst = edge_index[1]

    s0, cnt16 = _sc_pass0(edge_features, dst)
    x = _tc_fin0(s0, cnt16)

    for (w, b) in [(w2, b2), (w3, b3), (w4, b4), (w5, b5), (w6, b6), (w7, b7),
                   (w8, b8), (w9, b9)]:
        xs = x.reshape(NC * NPAD, H)
        t = _sc_seg(xs, src, dst)
        x = _tc_layer(x, t, cnt16, w, b)

    y1l, y1r, y2l, y2r = _tc_final(x, we, be)
    ef = _sc_edge(y1l, y1r, y2l, y2r, src, dst)
    side_loss = jnp.float32(0.0)
    return (ef, side_loss)
